# baseline (speedup 1.0000x reference)
import functools

import jax
import jax.numpy as jnp
from jax import lax
from jax.experimental import pallas as pl
from jax.experimental.pallas import tpu as pltpu
from jax.experimental.pallas import tpu_sc as plsc

_NNODES = 4096
_FEAT = 10
_DIM = 10
_K = 8
_ALPHA = 3.0
_T = 512

_NC = 2
_NS = 16
_NW = _NC * _NS
_NROWS = 2 * _NNODES
_RPW = _NROWS // _NW
_G = 8
_NG = _RPW // _G
_GW = _G * _NNODES


def _topk_body(x_ref, w1_ref, b1_ref, w2_ref, b2_ref, vals_ref, idx_ref,
               nv1_ref, nv2_ref):
    t = pl.program_id(1)

    @pl.when(t == 0)
    def _():
        x = x_ref[0]
        nv1_ref[...] = jnp.tanh(
            _ALPHA * (jnp.dot(x, w1_ref[...], preferred_element_type=jnp.float32)
                      + b1_ref[0][None, :]))
        nv2_ref[...] = jnp.tanh(
            _ALPHA * (jnp.dot(x, w2_ref[...], preferred_element_type=jnp.float32)
                      + b2_ref[0][None, :]))

    nv1 = nv1_ref[...]
    nv2 = nv2_ref[...]
    nv1_r = nv1_ref[pl.ds(t * _T, _T), :]
    nv2_r = nv2_ref[pl.ds(t * _T, _T), :]

    dn = (((1,), (1,)), ((), ()))
    a = (lax.dot_general(nv1_r, nv2, dn, preferred_element_type=jnp.float32)
         - lax.dot_general(nv2_r, nv1, dn, preferred_element_type=jnp.float32))
    adj = jnp.maximum(jnp.tanh(_ALPHA * a), 0.0)

    col = lax.broadcasted_iota(
        jnp.int32, (_T, _NNODES), 1).astype(jnp.float32)
    work = adj
    ms = []
    fs = []
    for _ in range(_K):
        m = jnp.max(work, axis=1, keepdims=True)
        idxs = jnp.where(work == m, col, 8192.0)
        first = jnp.min(idxs, axis=1, keepdims=True)
        work = jnp.where(col == first, -1.0, work)
        ms.append(m)
        fs.append(first)

    vals_ref[...] = jnp.concatenate(ms, axis=1)
    row8 = jnp.bitwise_and(
        lax.broadcasted_iota(jnp.int32, (_T, _K), 0), 7) * _NNODES
    idx_ref[...] = jnp.concatenate(fs, axis=1).astype(jnp.int32) + row8


def _tc_topk(X, W1, b1, W2, b2):
    B, N, F = X.shape
    nt = N // _T
    grid = (B, nt)
    return pl.pallas_call(
        _topk_body,
        grid=grid,
        in_specs=[
            pl.BlockSpec((1, N, F), lambda b, t: (b, 0, 0)),
            pl.BlockSpec((F, _DIM), lambda b, t: (0, 0)),
            pl.BlockSpec((1, _DIM), lambda b, t: (0, 0)),
            pl.BlockSpec((F, _DIM), lambda b, t: (0, 0)),
            pl.BlockSpec((1, _DIM), lambda b, t: (0, 0)),
        ],
        out_specs=[
            pl.BlockSpec((_T, _K), lambda b, t: (b * nt + t, 0)),
            pl.BlockSpec((_T, _K), lambda b, t: (b * nt + t, 0)),
        ],
        out_shape=[
            jax.ShapeDtypeStruct((B * N, _K), jnp.float32),
            jax.ShapeDtypeStruct((B * N, _K), jnp.int32),
        ],
        scratch_shapes=[
            pltpu.VMEM((N, _DIM), jnp.float32),
            pltpu.VMEM((N, _DIM), jnp.float32),
        ],
    )(X, W1, b1.reshape(1, -1), W2, b2.reshape(1, -1))


def _sc_body(vals_hbm, idx_hbm, zeros_hbm, out_hbm, idxv, valv, zv_ref,
             shared, sem0, sem1):
    c = lax.axis_index("c")
    s = lax.axis_index("s")
    wid = s * _NC + c
    base_row = wid * _RPW

    pltpu.sync_copy(idx_hbm.at[pl.ds(base_row * _K, _RPW * _K)], idxv)
    pltpu.sync_copy(vals_hbm.at[pl.ds(base_row * _K, _RPW * _K)], valv)
    sb0 = (s * 2) * _GW
    sb1 = (s * 2 + 1) * _GW
    pltpu.sync_copy(zeros_hbm, shared.at[pl.ds(sb0, _GW)])
    pltpu.sync_copy(zeros_hbm, shared.at[pl.ds(sb1, _GW)])
    pltpu.sync_copy(zeros_hbm.at[pl.ds(0, 16)], zv_ref)

    sbases = (sb0, sb1)
    sems = (sem0, sem1)

    def _dst(g):
        return out_hbm.at[pl.ds((base_row + g * _G) * _NNODES, _GW)]

    for g in range(_NG):
        sbase = sbases[g % 2]
        sem = sems[g % 2]
        if g >= 2:
            pltpu.make_async_copy(
                shared.at[pl.ds(sbase, _GW)], _dst(g - 2), sem).wait()
            for v in range(_K // 2):
                ci = idxv[pl.ds((g - 2) * (_G * _K) + v * 16, 16)]
                pltpu.sync_copy(zv_ref, shared.at[ci + sbase])
        for v in range(_K // 2):
            off = g * (_G * _K) + v * 16
            ci = idxv[pl.ds(off, 16)]
            pltpu.sync_copy(valv.at[pl.ds(off, 16)], shared.at[ci + sbase])
        pltpu.make_async_copy(shared.at[pl.ds(sbase, _GW)], _dst(g), sem).start()

    for g in (_NG - 2, _NG - 1):
        pltpu.make_async_copy(
            shared.at[pl.ds(sbases[g % 2], _GW)], _dst(g), sems[g % 2]).wait()


@jax.jit
def kernel(X, W1, b1, W2, b2):
    B, N, _ = X.shape
    vals, idx = _tc_topk(X, W1, b1, W2, b2)

    mesh = plsc.VectorSubcoreMesh(core_axis_name="c", subcore_axis_name="s")
    scatter = functools.partial(
        pl.kernel,
        mesh=mesh,
        out_type=jax.ShapeDtypeStruct((B * N * N,), jnp.float32),
        scratch_types=[
            pltpu.VMEM((_RPW * _K,), jnp.int32),
            pltpu.VMEM((_RPW * _K,), jnp.float32),
            pltpu.VMEM((16,), jnp.float32),
            pltpu.VMEM_SHARED((_NS * 2 * _GW,), jnp.float32),
            pltpu.SemaphoreType.DMA,
            pltpu.SemaphoreType.DMA,
        ],
    )(_sc_body)
    out_flat = scatter(vals.reshape(-1), idx.reshape(-1),
                       jnp.zeros((_GW,), jnp.float32))
    return out_flat.reshape(B, N, N)


# speedup vs baseline: 1.0024x; 1.0024x over previous
import functools

import jax
import jax.numpy as jnp
from jax import lax
from jax.experimental import pallas as pl
from jax.experimental.pallas import tpu as pltpu
from jax.experimental.pallas import tpu_sc as plsc

_NNODES = 4096
_FEAT = 10
_DIM = 10
_K = 8
_ALPHA = 3.0
_T = 512

_NC = 2
_NS = 16
_NW = _NC * _NS
_NROWS = 2 * _NNODES
_RPW = _NROWS // _NW
_G = 8
_NG = _RPW // _G
_GW = _G * _NNODES


def _topk_body(x_ref, w1_ref, b1_ref, w2_ref, b2_ref, vals_ref, idx_ref,
               nv1_ref, nv2_ref):
    t = pl.program_id(1)

    @pl.when(t == 0)
    def _():
        x = x_ref[0]
        nv1_ref[...] = jnp.tanh(
            _ALPHA * (jnp.dot(x, w1_ref[...], preferred_element_type=jnp.float32)
                      + b1_ref[0][None, :]))
        nv2_ref[...] = jnp.tanh(
            _ALPHA * (jnp.dot(x, w2_ref[...], preferred_element_type=jnp.float32)
                      + b2_ref[0][None, :]))

    nv1 = nv1_ref[...]
    nv2 = nv2_ref[...]
    nv1_r = nv1_ref[pl.ds(t * _T, _T), :]
    nv2_r = nv2_ref[pl.ds(t * _T, _T), :]

    dn = (((1,), (1,)), ((), ()))
    a = (lax.dot_general(nv1_r, nv2, dn, preferred_element_type=jnp.float32)
         - lax.dot_general(nv2_r, nv1, dn, preferred_element_type=jnp.float32))
    adj = jnp.maximum(jnp.tanh(_ALPHA * a), 0.0)

    col = lax.broadcasted_iota(
        jnp.int32, (_T, _NNODES), 1).astype(jnp.float32)
    work = adj
    ms = []
    fs = []
    for _ in range(_K):
        m = jnp.max(work, axis=1, keepdims=True)
        idxs = jnp.where(work == m, col, 8192.0)
        first = jnp.min(idxs, axis=1, keepdims=True)
        work = jnp.where(col == first, -1.0, work)
        ms.append(m)
        fs.append(first)

    vals_ref[...] = jnp.concatenate(ms, axis=1)
    row8 = jnp.bitwise_and(
        lax.broadcasted_iota(jnp.int32, (_T, _K), 0), 7) * _NNODES
    idx_ref[...] = jnp.concatenate(fs, axis=1).astype(jnp.int32) + row8


def _tc_topk(X, W1, b1, W2, b2):
    B, N, F = X.shape
    nt = N // _T
    grid = (B, nt)
    return pl.pallas_call(
        _topk_body,
        grid=grid,
        in_specs=[
            pl.BlockSpec((1, N, F), lambda b, t: (b, 0, 0)),
            pl.BlockSpec((F, _DIM), lambda b, t: (0, 0)),
            pl.BlockSpec((1, _DIM), lambda b, t: (0, 0)),
            pl.BlockSpec((F, _DIM), lambda b, t: (0, 0)),
            pl.BlockSpec((1, _DIM), lambda b, t: (0, 0)),
        ],
        out_specs=[
            pl.BlockSpec((_T, _K), lambda b, t: (b * nt + t, 0)),
            pl.BlockSpec((_T, _K), lambda b, t: (b * nt + t, 0)),
        ],
        out_shape=[
            jax.ShapeDtypeStruct((B * N, _K), jnp.float32),
            jax.ShapeDtypeStruct((B * N, _K), jnp.int32),
        ],
        scratch_shapes=[
            pltpu.VMEM((N, _DIM), jnp.float32),
            pltpu.VMEM((N, _DIM), jnp.float32),
        ],
    )(X, W1, b1.reshape(1, -1), W2, b2.reshape(1, -1))


def _sc_body(vals_hbm, idx_hbm, zeros_hbm, out_hbm, idxv, valv, zv_ref,
             shared, sem0, sem1, ssem):
    c = lax.axis_index("c")
    s = lax.axis_index("s")
    wid = s * _NC + c
    base_row = wid * _RPW

    pltpu.sync_copy(idx_hbm.at[pl.ds(base_row * _K, _RPW * _K)], idxv)
    pltpu.sync_copy(vals_hbm.at[pl.ds(base_row * _K, _RPW * _K)], valv)
    sb0 = (s * 2) * _GW
    sb1 = (s * 2 + 1) * _GW
    pltpu.sync_copy(zeros_hbm, shared.at[pl.ds(sb0, _GW)])
    pltpu.sync_copy(zeros_hbm, shared.at[pl.ds(sb1, _GW)])
    pltpu.sync_copy(zeros_hbm.at[pl.ds(0, 16)], zv_ref)

    sbases = (sb0, sb1)
    sems = (sem0, sem1)

    def _dst(g):
        return out_hbm.at[pl.ds((base_row + g * _G) * _NNODES, _GW)]

    nv = _K // 2

    def _zscat(g, sbase):
        ci = [idxv[pl.ds(g * (_G * _K) + v * 16, 16)] for v in range(nv)]
        for v in range(nv):
            pltpu.make_async_copy(zv_ref, shared.at[ci[v] + sbase], ssem).start()
        for v in range(nv):
            pltpu.make_async_copy(zv_ref, shared.at[ci[v] + sbase], ssem).wait()

    def _vscat(g, sbase):
        offs = [g * (_G * _K) + v * 16 for v in range(nv)]
        ci = [idxv[pl.ds(o, 16)] for o in offs]
        for v in range(nv):
            pltpu.make_async_copy(
                valv.at[pl.ds(offs[v], 16)], shared.at[ci[v] + sbase], ssem).start()
        for v in range(nv):
            pltpu.make_async_copy(
                valv.at[pl.ds(offs[v], 16)], shared.at[ci[v] + sbase], ssem).wait()

    for g in range(_NG):
        sbase = sbases[g % 2]
        sem = sems[g % 2]
        if g >= 2:
            pltpu.make_async_copy(
                shared.at[pl.ds(sbase, _GW)], _dst(g - 2), sem).wait()
            _zscat(g - 2, sbase)
        _vscat(g, sbase)
        pltpu.make_async_copy(shared.at[pl.ds(sbase, _GW)], _dst(g), sem).start()

    for g in (_NG - 2, _NG - 1):
        pltpu.make_async_copy(
            shared.at[pl.ds(sbases[g % 2], _GW)], _dst(g), sems[g % 2]).wait()


@jax.jit
def kernel(X, W1, b1, W2, b2):
    B, N, _ = X.shape
    vals, idx = _tc_topk(X, W1, b1, W2, b2)

    mesh = plsc.VectorSubcoreMesh(core_axis_name="c", subcore_axis_name="s")
    scatter = functools.partial(
        pl.kernel,
        mesh=mesh,
        out_type=jax.ShapeDtypeStruct((B * N * N,), jnp.float32),
        scratch_types=[
            pltpu.VMEM((_RPW * _K,), jnp.int32),
            pltpu.VMEM((_RPW * _K,), jnp.float32),
            pltpu.VMEM((16,), jnp.float32),
            pltpu.VMEM_SHARED((_NS * 2 * _GW,), jnp.float32),
            pltpu.SemaphoreType.DMA,
            pltpu.SemaphoreType.DMA,
            pltpu.SemaphoreType.DMA,
        ],
    )(_sc_body)
    out_flat = scatter(vals.reshape(-1), idx.reshape(-1),
                       jnp.zeros((_GW,), jnp.float32))
    return out_flat.reshape(B, N, N)


# speedup vs baseline: 1.7609x; 1.7566x over previous
import functools

import jax
import jax.numpy as jnp
from jax.experimental import pallas as pl
from jax.experimental.pallas import tpu as pltpu

_NNODES = 4096
_FEAT = 10
_DIM = 10
_K = 8
_ALPHA = 3.0
_T = 512


def _body(x_ref, w1_ref, b1_ref, w2_ref, b2_ref, out_ref, nv1_ref, nv2_ref):
    t = pl.program_id(1)

    @pl.when(t == 0)
    def _():
        x = x_ref[0]
        nv1_ref[...] = jnp.tanh(
            _ALPHA * (jnp.dot(x, w1_ref[...], preferred_element_type=jnp.float32)
                      + b1_ref[0][None, :]))
        nv2_ref[...] = jnp.tanh(
            _ALPHA * (jnp.dot(x, w2_ref[...], preferred_element_type=jnp.float32)
                      + b2_ref[0][None, :]))

    nv1 = nv1_ref[...]
    nv2 = nv2_ref[...]
    nv1_r = nv1_ref[pl.ds(t * _T, _T), :]
    nv2_r = nv2_ref[pl.ds(t * _T, _T), :]

    dn = (((1,), (1,)), ((), ()))
    a = (jax.lax.dot_general(nv1_r, nv2, dn, preferred_element_type=jnp.float32)
         - jax.lax.dot_general(nv2_r, nv1, dn, preferred_element_type=jnp.float32))
    adj = jnp.maximum(jnp.tanh(_ALPHA * a), 0.0)

    col = jax.lax.broadcasted_iota(
        jnp.int32, (_T, _NNODES), 1).astype(jnp.float32)
    work = adj
    for _ in range(_K):
        m = jnp.max(work, axis=1, keepdims=True)
        idxs = jnp.where(work == m, col, 8192.0)
        first = jnp.min(idxs, axis=1, keepdims=True)
        work = jnp.where(col == first, -1.0, work)

    out_ref[0] = jnp.where(work < 0.0, adj, 0.0)


@jax.jit
def kernel(X, W1, b1, W2, b2):
    B, N, F = X.shape
    grid = (B, N // _T)
    return pl.pallas_call(
        _body,
        grid=grid,
        in_specs=[
            pl.BlockSpec((1, N, F), lambda b, t: (b, 0, 0)),
            pl.BlockSpec((F, _DIM), lambda b, t: (0, 0)),
            pl.BlockSpec((1, _DIM), lambda b, t: (0, 0)),
            pl.BlockSpec((F, _DIM), lambda b, t: (0, 0)),
            pl.BlockSpec((1, _DIM), lambda b, t: (0, 0)),
        ],
        out_specs=pl.BlockSpec((1, _T, N), lambda b, t: (b, t, 0)),
        out_shape=jax.ShapeDtypeStruct((B, N, N), jnp.float32),
        scratch_shapes=[
            pltpu.VMEM((N, _DIM), jnp.float32),
            pltpu.VMEM((N, _DIM), jnp.float32),
        ],
    )(X, W1, b1.reshape(1, -1), W2, b2.reshape(1, -1))
